# Initial kernel scaffold; baseline (speedup 1.0000x reference)
import jax
import jax.numpy as jnp
from jax.experimental import pallas as pl


def kernel(input_features, neighbor_inds, weightnet, additional_features, linear_weight, linear_bias):
    raise NotImplementedError("write your pallas kernel here")


# speedup vs baseline: 18.2834x; 18.2834x over previous
import functools

import jax
import jax.numpy as jnp
import numpy as np
from jax import lax
from jax.experimental import pallas as pl
from jax.experimental.pallas import tpu as pltpu
from jax.experimental.pallas import tpu_sc as plsc

_NC = 2
_NS = 16
_NW = _NC * _NS


def _sc_gather(table, idx, chunk):
    rows, cols = table.shape
    total = idx.shape[0]
    n_chunks = total // chunk
    cpw = n_chunks // _NW
    idx2 = idx.reshape(_NW, cpw, chunk)
    mesh = plsc.VectorSubcoreMesh(core_axis_name="c", subcore_axis_name="s")

    @functools.partial(
        pl.kernel,
        mesh=mesh,
        out_type=jax.ShapeDtypeStruct((total, cols), jnp.float32),
        scratch_types=[
            pltpu.VMEM((cpw, chunk), jnp.int32),
            pltpu.VMEM((chunk, cols), jnp.float32),
            pltpu.SemaphoreType.DMA,
        ],
        compiler_params=pltpu.CompilerParams(use_tc_tiling_on_sc=False),
    )
    def gather_kernel(table_hbm, idx_hbm, out_hbm, idx_v, rows_v, sem):
        wid = lax.axis_index("s") * _NC + lax.axis_index("c")
        base = wid * cpw
        pltpu.sync_copy(idx_hbm.at[wid], idx_v)

        def body(i, carry):
            pltpu.async_copy(table_hbm.at[idx_v.at[i]], rows_v, sem).wait()
            off = pl.multiple_of((base + i) * chunk, 8)
            pltpu.sync_copy(rows_v, out_hbm.at[pl.ds(off, chunk)])
            return carry

        lax.fori_loop(0, cpw, body, 0)

    return gather_kernel(table, idx2)


def _tc_body(g_ref, w_ref, a_ref, wgt_ref, wat_ref, b_ref, r16_ref, s16_ref,
             r4_ref, s4_ref, o_ref, *, K, C_IN, C_MID, C_ADD):
    t = g_ref.shape[0]
    r16 = r16_ref[...]
    s16 = s16_ref[...]
    r4 = r4_ref[...]
    s4 = s4_ref[...]
    pc_g = jnp.zeros((t, C_IN * C_MID), jnp.float32)
    pc_a = jnp.zeros((t, C_ADD * C_MID), jnp.float32)
    for k in range(K):
        gk = g_ref[:, k * C_IN:(k + 1) * C_IN]
        wk = w_ref[:, k * C_MID:(k + 1) * C_MID]
        ak = a_ref[:, k * C_ADD:(k + 1) * C_ADD]
        g_rep = jnp.dot(gk, r16, preferred_element_type=jnp.float32)
        w_til = jnp.dot(wk, s16, preferred_element_type=jnp.float32)
        pc_g = pc_g + g_rep * w_til
        a_rep = jnp.dot(ak, r4, preferred_element_type=jnp.float32)
        w_til4 = jnp.dot(wk, s4, preferred_element_type=jnp.float32)
        pc_a = pc_a + a_rep * w_til4
    out = jnp.dot(pc_g, wgt_ref[...], preferred_element_type=jnp.float32)
    out = out + jnp.dot(pc_a, wat_ref[...], preferred_element_type=jnp.float32)
    o_ref[...] = out + b_ref[...]


def _tc_fused(gathered2, wn2, add2, wg_t, wa_t, bias2, *, K, C_IN, C_MID,
              C_ADD, OUT_F, tile, interpret=False):
    bn = gathered2.shape[0]
    grid = (bn // tile,)
    cg = C_IN * C_MID
    ca = C_ADD * C_MID
    r16 = jnp.asarray(np.repeat(np.eye(C_IN, dtype=np.float32), C_MID, axis=1))
    s16 = jnp.asarray(np.tile(np.eye(C_MID, dtype=np.float32), (1, C_IN)))
    r4 = jnp.asarray(np.repeat(np.eye(C_ADD, dtype=np.float32), C_MID, axis=1))
    s4 = jnp.asarray(np.tile(np.eye(C_MID, dtype=np.float32), (1, C_ADD)))
    body = functools.partial(_tc_body, K=K, C_IN=C_IN, C_MID=C_MID, C_ADD=C_ADD)
    zero = lambda i: (0, 0)
    return pl.pallas_call(
        body,
        grid=grid,
        in_specs=[
            pl.BlockSpec((tile, K * C_IN), lambda i: (i, 0)),
            pl.BlockSpec((tile, K * C_MID), lambda i: (i, 0)),
            pl.BlockSpec((tile, K * C_ADD), lambda i: (i, 0)),
            pl.BlockSpec((cg, OUT_F), zero),
            pl.BlockSpec((ca, OUT_F), zero),
            pl.BlockSpec((1, OUT_F), zero),
            pl.BlockSpec((C_IN, cg), zero),
            pl.BlockSpec((C_MID, cg), zero),
            pl.BlockSpec((C_ADD, ca), zero),
            pl.BlockSpec((C_MID, ca), zero),
        ],
        out_specs=pl.BlockSpec((tile, OUT_F), lambda i: (i, 0)),
        out_shape=jax.ShapeDtypeStruct((bn, OUT_F), jnp.float32),
        interpret=interpret,
    )(gathered2, wn2, add2, wg_t, wa_t, bias2, r16, s16, r4, s4)


def kernel(input_features, neighbor_inds, weightnet, additional_features,
           linear_weight, linear_bias):
    b, n, c_in = input_features.shape
    _, _, k = neighbor_inds.shape
    c_mid = weightnet.shape[-1]
    c_add = additional_features.shape[-1]
    out_f = linear_weight.shape[0]
    bn = b * n

    table = input_features.reshape(bn, c_in)
    offs = (jnp.arange(b, dtype=neighbor_inds.dtype) * n)[:, None, None]
    idx = (neighbor_inds + offs).reshape(bn * k)

    gathered = _sc_gather(table, idx, chunk=80)

    wn2 = weightnet.reshape(bn, k * c_mid)
    add2 = additional_features.reshape(bn, k * c_add)
    wg_t = linear_weight[:, :c_in * c_mid].T
    wa_t = linear_weight[:, c_in * c_mid:].T
    bias2 = linear_bias.reshape(1, out_f)

    out = _tc_fused(gathered.reshape(bn, k * c_in), wn2, add2, wg_t, wa_t,
                    bias2, K=k, C_IN=c_in, C_MID=c_mid, C_ADD=c_add,
                    OUT_F=out_f, tile=2000)
    return out.reshape(b, n, out_f)
